# baseline (speedup 1.0000x reference)
import functools

import jax
import jax.numpy as jnp
from jax import lax
from jax.experimental import pallas as pl
from jax.experimental.pallas import tpu as pltpu
from jax.experimental.pallas import tpu_sc as plsc

B, H, T = 8, 1024, 4096
CD, CS = 8, 1024
TT = 2048
BT = B * T

_INFO = plsc.get_sparse_core_info()
_NC, _NS = _INFO.num_cores, _INFO.num_subcores
_NW = _NC * _NS
_BPW = BT // _NW


def _proj_argmax_kernel(h_ref, w_in_ref, b_in_ref, cb_ref,
                        idx_ref, proj_ref):
    h = h_ref[0]
    cb = cb_ref[...]

    p = lax.dot_general(w_in_ref[...], h, (((1,), (0,)), ((), ())),
                        preferred_element_type=jnp.float32)
    p = p + b_in_ref[...][:, None]
    proj_ref[0] = p

    n = jnp.sqrt(jnp.sum(p * p, axis=0, keepdims=True))
    en = p / jnp.maximum(n, 1e-12)
    cbn = jnp.sqrt(jnp.sum(cb * cb, axis=1, keepdims=True))
    cn = cb / jnp.maximum(cbn, 1e-12)

    l2 = jnp.sum(en * en, axis=0, keepdims=True)
    cn2 = jnp.sum(cn * cn, axis=1, keepdims=True)
    sc = lax.dot_general(cn, en, (((1,), (0,)), ((), ())),
                         preferred_element_type=jnp.float32)
    dist = -(l2 - 2.0 * sc) + cn2

    idx_ref[0, 0] = jnp.argmax(dist, axis=0).astype(jnp.int32)


@functools.partial(
    pl.kernel,
    mesh=plsc.VectorSubcoreMesh(core_axis_name="c", subcore_axis_name="s"),
    out_type=[
        jax.ShapeDtypeStruct((CD, BT), jnp.float32),
        jax.ShapeDtypeStruct((B, T), jnp.int32),
        jax.ShapeDtypeStruct((_NW, 16), jnp.float32),
    ],
    scratch_types=[
        pltpu.VMEM((_BPW,), jnp.int32),
        pltpu.VMEM((CS * CD,), jnp.float32),
        pltpu.VMEM((CD, _BPW), jnp.float32),
        pltpu.VMEM((CD, _BPW), jnp.float32),
        pltpu.VMEM((16,), jnp.float32),
    ],
    compiler_params=pltpu.CompilerParams(needs_layout_passes=False),
)
def _sc_gather(cb_hbm, idx_hbm, proj_hbm, q_hbm, idx2_hbm, sse_hbm,
               idx_v, cb_v, p_v, q_v, acc_v):
    wid = lax.axis_index("s") * _NC + lax.axis_index("c")
    base = wid * _BPW
    bb = base // T
    t0 = base - bb * T
    pltpu.sync_copy(idx_hbm.at[bb, 0, pl.ds(t0, _BPW)], idx_v)
    pltpu.sync_copy(cb_hbm, cb_v)
    pltpu.sync_copy(proj_hbm.at[bb, :, pl.ds(t0, _BPW)], p_v)

    lanes = lax.iota(jnp.int32, 16)
    hi = lax.shift_right_logical(lanes, 3)
    lo = lax.bitwise_and(lanes, 7)

    def body(j, acc):
        tpos = hi + 2 * j
        rows = plsc.load_gather(idx_v, [tpos])
        eidx = lax.shift_left(rows, 3) + lo
        vals = plsc.load_gather(cb_v, [eidx])
        plsc.store_scatter(q_v, [lo, tpos], vals)
        pvals = plsc.load_gather(p_v, [lo, tpos])
        d = pvals - vals
        return acc + d * d

    acc = jax.lax.fori_loop(0, _BPW // 2, body,
                            jnp.zeros((16,), jnp.float32), unroll=8)
    acc_v[...] = acc
    pltpu.sync_copy(q_v, q_hbm.at[:, pl.ds(base, _BPW)])
    pltpu.sync_copy(idx_v, idx2_hbm.at[bb, pl.ds(t0, _BPW)])
    pltpu.sync_copy(acc_v, sse_hbm.at[wid])


def _out_loss_kernel(q_ref, w_out_ref, b_out_ref, sse_ref, out_ref, loss_ref):
    o = lax.dot_general(w_out_ref[...], q_ref[...], (((1,), (0,)), ((), ())),
                        preferred_element_type=jnp.float32)
    out_ref[0] = o + b_out_ref[...][:, None]

    @pl.when(pl.program_id(0) == 0)
    def _loss():
        loss_ref[0, 0] = jnp.sum(sse_ref[...]) / (B * CD * T)


@jax.jit
def _vq(hidden_state, W_in, b_in, codebook, W_out, b_out):
    idx3, proj = pl.pallas_call(
        _proj_argmax_kernel,
        grid=(B, T // TT),
        in_specs=[
            pl.BlockSpec((1, H, TT), lambda b, t: (b, 0, t)),
            pl.BlockSpec((CD, H), lambda b, t: (0, 0)),
            pl.BlockSpec((CD,), lambda b, t: (0,)),
            pl.BlockSpec((CS, CD), lambda b, t: (0, 0)),
        ],
        out_specs=[
            pl.BlockSpec((1, 1, TT), lambda b, t: (b, 0, t)),
            pl.BlockSpec((1, CD, TT), lambda b, t: (b, 0, t)),
        ],
        out_shape=[
            jax.ShapeDtypeStruct((B, 1, T), jnp.int32),
            jax.ShapeDtypeStruct((B, CD, T), jnp.float32),
        ],
    )(hidden_state, W_in, b_in, codebook)

    qT, idx2, sse_parts = _sc_gather(codebook.reshape(CS * CD), idx3, proj)

    out, loss2 = pl.pallas_call(
        _out_loss_kernel,
        grid=(B,),
        in_specs=[
            pl.BlockSpec((CD, T), lambda b: (0, b)),
            pl.BlockSpec((H, CD), lambda b: (0, 0)),
            pl.BlockSpec((H,), lambda b: (0,)),
            pl.BlockSpec((_NW, 16), lambda b: (0, 0)),
        ],
        out_specs=[
            pl.BlockSpec((1, H, T), lambda b: (b, 0, 0)),
            pl.BlockSpec(memory_space=pltpu.SMEM, block_shape=(1, 1),
                         index_map=lambda b: (0, 0)),
        ],
        out_shape=[
            jax.ShapeDtypeStruct((B, H, T), jnp.float32),
            jax.ShapeDtypeStruct((1, 1), jnp.float32),
        ],
    )(qT, W_out, b_out, sse_parts)

    loss = loss2[0, 0]
    return out, loss, loss, idx2, proj


def kernel(hidden_state, W_in, b_in, codebook, W_out, b_out):
    return _vq(hidden_state, W_in, b_in, codebook, W_out, b_out)


# speedup vs baseline: 1.0371x; 1.0371x over previous
import functools

import jax
import jax.numpy as jnp
from jax import lax
from jax.experimental import pallas as pl
from jax.experimental.pallas import tpu as pltpu
from jax.experimental.pallas import tpu_sc as plsc

B, H, T = 8, 1024, 4096
CD, CS = 8, 1024
TT = 2048
TTB = 2048
BT = B * T

_INFO = plsc.get_sparse_core_info()
_NC, _NS = _INFO.num_cores, _INFO.num_subcores
_NW = _NC * _NS
_BPW = BT // _NW


def _proj_argmax_kernel(h_ref, w_in_ref, b_in_ref, cb_ref,
                        idx_ref, proj_ref):
    h = h_ref[0]
    cb = cb_ref[...]

    p = lax.dot_general(w_in_ref[...], h, (((1,), (0,)), ((), ())),
                        preferred_element_type=jnp.float32)
    p = p + b_in_ref[...][:, None]
    proj_ref[0] = p

    n = jnp.sqrt(jnp.sum(p * p, axis=0, keepdims=True))
    en = p / jnp.maximum(n, 1e-12)
    cbn = jnp.sqrt(jnp.sum(cb * cb, axis=1, keepdims=True))
    cn = cb / jnp.maximum(cbn, 1e-12)

    l2 = jnp.sum(en * en, axis=0, keepdims=True)
    cn2 = jnp.sum(cn * cn, axis=1, keepdims=True)
    sc = lax.dot_general(cn, en, (((1,), (0,)), ((), ())),
                         preferred_element_type=jnp.float32)
    dist = -(l2 - 2.0 * sc) + cn2

    idx_ref[0, 0] = jnp.argmax(dist, axis=0).astype(jnp.int32)


@functools.partial(
    pl.kernel,
    mesh=plsc.VectorSubcoreMesh(core_axis_name="c", subcore_axis_name="s"),
    out_type=[
        jax.ShapeDtypeStruct((CD, BT), jnp.float32),
        jax.ShapeDtypeStruct((B, T), jnp.int32),
    ],
    scratch_types=[
        pltpu.VMEM((_BPW,), jnp.int32),
        pltpu.VMEM((CS * CD,), jnp.float32),
        pltpu.VMEM((CD, _BPW), jnp.float32),
    ],
    compiler_params=pltpu.CompilerParams(needs_layout_passes=False),
)
def _sc_gather(cb_hbm, idx_hbm, q_hbm, idx2_hbm, idx_v, cb_v, q_v):
    wid = lax.axis_index("s") * _NC + lax.axis_index("c")
    base = wid * _BPW
    bb = base // T
    t0 = base - bb * T
    pltpu.sync_copy(idx_hbm.at[bb, 0, pl.ds(t0, _BPW)], idx_v)
    pltpu.sync_copy(cb_hbm, cb_v)

    lanes = lax.iota(jnp.int32, 16)
    hi = lax.shift_right_logical(lanes, 3)
    lo = lax.bitwise_and(lanes, 7)

    def body(j, _):
        tpos = hi + 2 * j
        rows = plsc.load_gather(idx_v, [tpos])
        eidx = lax.shift_left(rows, 3) + lo
        vals = plsc.load_gather(cb_v, [eidx])
        plsc.store_scatter(q_v, [lo, tpos], vals)
        return _

    jax.lax.fori_loop(0, _BPW // 2, body, None, unroll=8)
    pltpu.sync_copy(q_v, q_hbm.at[:, pl.ds(base, _BPW)])
    pltpu.sync_copy(idx_v, idx2_hbm.at[bb, pl.ds(t0, _BPW)])


def _out_loss_kernel(q_ref, proj_ref, w_out_ref, b_out_ref,
                     out_ref, loss_ref):
    qb = q_ref[...]
    o = lax.dot_general(w_out_ref[...], qb, (((1,), (0,)), ((), ())),
                        preferred_element_type=jnp.float32)
    out_ref[0] = o + b_out_ref[...][:, None]

    d = proj_ref[0] - qb
    sse = jnp.sum(d * d)

    @pl.when(jnp.logical_and(pl.program_id(0) == 0, pl.program_id(1) == 0))
    def _init():
        loss_ref[0, 0] = 0.0

    loss_ref[0, 0] += sse / (B * CD * T)


@jax.jit
def _vq(hidden_state, W_in, b_in, codebook, W_out, b_out):
    idx3, proj = pl.pallas_call(
        _proj_argmax_kernel,
        grid=(B, T // TT),
        in_specs=[
            pl.BlockSpec((1, H, TT), lambda b, t: (b, 0, t)),
            pl.BlockSpec((CD, H), lambda b, t: (0, 0)),
            pl.BlockSpec((CD,), lambda b, t: (0,)),
            pl.BlockSpec((CS, CD), lambda b, t: (0, 0)),
        ],
        out_specs=[
            pl.BlockSpec((1, 1, TT), lambda b, t: (b, 0, t)),
            pl.BlockSpec((1, CD, TT), lambda b, t: (b, 0, t)),
        ],
        out_shape=[
            jax.ShapeDtypeStruct((B, 1, T), jnp.int32),
            jax.ShapeDtypeStruct((B, CD, T), jnp.float32),
        ],
    )(hidden_state, W_in, b_in, codebook)

    qT, idx2 = _sc_gather(codebook.reshape(CS * CD), idx3)

    out, loss2 = pl.pallas_call(
        _out_loss_kernel,
        grid=(B, T // TTB),
        in_specs=[
            pl.BlockSpec((CD, TTB), lambda b, t: (0, b * (T // TTB) + t)),
            pl.BlockSpec((1, CD, TTB), lambda b, t: (b, 0, t)),
            pl.BlockSpec((H, CD), lambda b, t: (0, 0)),
            pl.BlockSpec((H,), lambda b, t: (0,)),
        ],
        out_specs=[
            pl.BlockSpec((1, H, TTB), lambda b, t: (b, 0, t)),
            pl.BlockSpec(memory_space=pltpu.SMEM, block_shape=(1, 1),
                         index_map=lambda b, t: (0, 0)),
        ],
        out_shape=[
            jax.ShapeDtypeStruct((B, H, T), jnp.float32),
            jax.ShapeDtypeStruct((1, 1), jnp.float32),
        ],
    )(qT, proj, W_out, b_out)

    loss = loss2[0, 0]
    return out, loss, loss, idx2, proj


def kernel(hidden_state, W_in, b_in, codebook, W_out, b_out):
    return _vq(hidden_state, W_in, b_in, codebook, W_out, b_out)


# speedup vs baseline: 1.0546x; 1.0168x over previous
import functools

import jax
import jax.numpy as jnp
from jax import lax
from jax.experimental import pallas as pl
from jax.experimental.pallas import tpu as pltpu
from jax.experimental.pallas import tpu_sc as plsc

B, H, T = 8, 1024, 4096
CD, CS = 8, 1024
TT = 2048
TTB = 2048
BT = B * T

_INFO = plsc.get_sparse_core_info()
_NC, _NS = _INFO.num_cores, _INFO.num_subcores
_NW = _NC * _NS
_BPW = BT // _NW


def _proj_argmax_kernel(h_ref, w_in_ref, b_in_ref, cb_ref,
                        idx_ref, proj_ref):
    h = h_ref[0]
    cb = cb_ref[...]

    p = lax.dot_general(w_in_ref[...], h, (((1,), (0,)), ((), ())),
                        preferred_element_type=jnp.float32)
    p = p + b_in_ref[...][:, None]
    proj_ref[0] = p

    n = jnp.sqrt(jnp.sum(p * p, axis=0, keepdims=True))
    en = p / jnp.maximum(n, 1e-12)
    cbn = jnp.sqrt(jnp.sum(cb * cb, axis=1, keepdims=True))
    cn = cb / jnp.maximum(cbn, 1e-12)

    l2 = jnp.sum(en * en, axis=0, keepdims=True)
    cn2 = jnp.sum(cn * cn, axis=1, keepdims=True)
    sc = lax.dot_general(cn, en, (((1,), (0,)), ((), ())),
                         preferred_element_type=jnp.float32)
    dist = (2.0 * sc - l2) + cn2

    idx_ref[0, 0] = jnp.argmax(dist, axis=0).astype(jnp.int32)


@functools.partial(
    pl.kernel,
    mesh=plsc.VectorSubcoreMesh(core_axis_name="c", subcore_axis_name="s"),
    out_type=[
        jax.ShapeDtypeStruct((CD, BT), jnp.float32),
        jax.ShapeDtypeStruct((B, T), jnp.int32),
    ],
    scratch_types=[
        pltpu.VMEM((_BPW,), jnp.int32),
        pltpu.VMEM((CS * CD,), jnp.float32),
        pltpu.VMEM((CD, _BPW), jnp.float32),
    ],
    compiler_params=pltpu.CompilerParams(needs_layout_passes=False),
)
def _sc_gather(cb_hbm, idx_hbm, q_hbm, idx2_hbm, idx_v, cb_v, q_v):
    wid = lax.axis_index("s") * _NC + lax.axis_index("c")
    base = wid * _BPW
    bb = base // T
    t0 = base - bb * T
    pltpu.sync_copy(idx_hbm.at[bb, 0, pl.ds(t0, _BPW)], idx_v)
    pltpu.sync_copy(cb_hbm, cb_v)

    lanes = lax.iota(jnp.int32, 16)
    hi = lax.shift_right_logical(lanes, 3)
    lo = lax.bitwise_and(lanes, 7)

    def body(j, _):
        tpos = hi + 2 * j
        rows = plsc.load_gather(idx_v, [tpos])
        eidx = lax.shift_left(rows, 3) + lo
        vals = plsc.load_gather(cb_v, [eidx])
        plsc.store_scatter(q_v, [lo, tpos], vals)
        return _

    jax.lax.fori_loop(0, _BPW // 2, body, None, unroll=8)
    pltpu.sync_copy(q_v, q_hbm.at[:, pl.ds(base, _BPW)])
    pltpu.sync_copy(idx_v, idx2_hbm.at[bb, pl.ds(t0, _BPW)])


def _out_loss_kernel(q_ref, proj_ref, w_out_ref, b_out_ref,
                     out_ref, loss_ref):
    qb = q_ref[...]
    o = lax.dot_general(w_out_ref[...], qb, (((1,), (0,)), ((), ())),
                        preferred_element_type=jnp.float32)
    out_ref[0] = o + b_out_ref[...][:, None]

    d = proj_ref[0] - qb
    sse = jnp.sum(d * d)

    @pl.when(jnp.logical_and(pl.program_id(0) == 0, pl.program_id(1) == 0))
    def _init():
        loss_ref[0, 0] = 0.0

    loss_ref[0, 0] += sse / (B * CD * T)


@jax.jit
def _vq(hidden_state, W_in, b_in, codebook, W_out, b_out):
    idx3, proj = pl.pallas_call(
        _proj_argmax_kernel,
        grid=(B, T // TT),
        in_specs=[
            pl.BlockSpec((1, H, TT), lambda b, t: (b, 0, t)),
            pl.BlockSpec((CD, H), lambda b, t: (0, 0)),
            pl.BlockSpec((CD,), lambda b, t: (0,)),
            pl.BlockSpec((CS, CD), lambda b, t: (0, 0)),
        ],
        out_specs=[
            pl.BlockSpec((1, 1, TT), lambda b, t: (b, 0, t)),
            pl.BlockSpec((1, CD, TT), lambda b, t: (b, 0, t)),
        ],
        out_shape=[
            jax.ShapeDtypeStruct((B, 1, T), jnp.int32),
            jax.ShapeDtypeStruct((B, CD, T), jnp.float32),
        ],
    )(hidden_state, W_in, b_in, codebook)

    qT, idx2 = _sc_gather(codebook.reshape(CS * CD), idx3)

    out, loss2 = pl.pallas_call(
        _out_loss_kernel,
        grid=(B, T // TTB),
        in_specs=[
            pl.BlockSpec((CD, TTB), lambda b, t: (0, b * (T // TTB) + t)),
            pl.BlockSpec((1, CD, TTB), lambda b, t: (b, 0, t)),
            pl.BlockSpec((H, CD), lambda b, t: (0, 0)),
            pl.BlockSpec((H,), lambda b, t: (0,)),
        ],
        out_specs=[
            pl.BlockSpec((1, H, TTB), lambda b, t: (b, 0, t)),
            pl.BlockSpec(memory_space=pltpu.SMEM, block_shape=(1, 1),
                         index_map=lambda b, t: (0, 0)),
        ],
        out_shape=[
            jax.ShapeDtypeStruct((B, H, T), jnp.float32),
            jax.ShapeDtypeStruct((1, 1), jnp.float32),
        ],
    )(qT, proj, W_out, b_out)

    loss = loss2[0, 0]
    return out, loss, loss, idx2, proj


def kernel(hidden_state, W_in, b_in, codebook, W_out, b_out):
    return _vq(hidden_state, W_in, b_in, codebook, W_out, b_out)


# speedup vs baseline: 1.0568x; 1.0021x over previous
import functools

import jax
import jax.numpy as jnp
from jax import lax
from jax.experimental import pallas as pl
from jax.experimental.pallas import tpu as pltpu
from jax.experimental.pallas import tpu_sc as plsc

B, H, T = 8, 1024, 4096
CD, CS = 8, 1024
TT = 2048
TTB = 2048
BT = B * T

_INFO = plsc.get_sparse_core_info()
_NC, _NS = _INFO.num_cores, _INFO.num_subcores
_NW = _NC * _NS
_BPW = BT // _NW


def _proj_argmax_kernel(h_ref, w_in_ref, b_in_ref, cb_ref,
                        idx_ref, proj_ref):
    h = h_ref[0]
    cb = cb_ref[...]

    p = lax.dot_general(w_in_ref[...], h, (((1,), (0,)), ((), ())),
                        preferred_element_type=jnp.float32)
    p = p + b_in_ref[...][:, None]
    proj_ref[0] = p

    n = jnp.sqrt(jnp.sum(p * p, axis=0, keepdims=True))
    en = p / jnp.maximum(n, 1e-12)
    cbn = jnp.sqrt(jnp.sum(cb * cb, axis=1, keepdims=True))
    cn = cb / jnp.maximum(cbn, 1e-12)

    l2 = jnp.sum(en * en, axis=0, keepdims=True)
    cn2 = jnp.sum(cn * cn, axis=1, keepdims=True)
    sc = lax.dot_general(cn, en, (((1,), (0,)), ((), ())),
                         preferred_element_type=jnp.float32)
    dist = (2.0 * sc - l2) + cn2

    idx_ref[0, 0] = jnp.argmax(dist, axis=0).astype(jnp.int32)


@functools.partial(
    pl.kernel,
    mesh=plsc.VectorSubcoreMesh(core_axis_name="c", subcore_axis_name="s"),
    out_type=[
        jax.ShapeDtypeStruct((CD, BT), jnp.float32),
        jax.ShapeDtypeStruct((B, T), jnp.int32),
    ],
    scratch_types=[
        pltpu.VMEM((_BPW,), jnp.int32),
        pltpu.VMEM((CS * CD,), jnp.float32),
        pltpu.VMEM((CD, _BPW), jnp.float32),
    ],
    compiler_params=pltpu.CompilerParams(needs_layout_passes=False),
)
def _sc_gather(cb_hbm, idx_hbm, q_hbm, idx2_hbm, idx_v, cb_v, q_v):
    wid = lax.axis_index("s") * _NC + lax.axis_index("c")
    base = wid * _BPW
    bb = base // T
    t0 = base - bb * T
    pltpu.sync_copy(idx_hbm.at[bb, 0, pl.ds(t0, _BPW)], idx_v)
    pltpu.sync_copy(cb_hbm, cb_v)

    lanes = lax.iota(jnp.int32, 16)
    hi = lax.shift_right_logical(lanes, 3)
    lo = lax.bitwise_and(lanes, 7)

    def body(j, _):
        tpos = hi + 2 * j
        rows = plsc.load_gather(idx_v, [tpos])
        eidx = lax.shift_left(rows, 3) + lo
        vals = plsc.load_gather(cb_v, [eidx])
        plsc.store_scatter(q_v, [lo, tpos], vals)
        return _

    jax.lax.fori_loop(0, _BPW // 2, body, None, unroll=16)
    pltpu.sync_copy(q_v, q_hbm.at[:, pl.ds(base, _BPW)])
    pltpu.sync_copy(idx_v, idx2_hbm.at[bb, pl.ds(t0, _BPW)])


def _out_loss_kernel(q_ref, proj_ref, w_out_ref, b_out_ref,
                     out_ref, loss_ref):
    qb = q_ref[...]
    o = lax.dot_general(w_out_ref[...], qb, (((1,), (0,)), ((), ())),
                        preferred_element_type=jnp.float32)
    out_ref[0] = o + b_out_ref[...][:, None]

    d = proj_ref[0] - qb
    sse = jnp.sum(d * d)

    @pl.when(jnp.logical_and(pl.program_id(0) == 0, pl.program_id(1) == 0))
    def _init():
        loss_ref[0, 0] = 0.0

    loss_ref[0, 0] += sse / (B * CD * T)


@jax.jit
def _vq(hidden_state, W_in, b_in, codebook, W_out, b_out):
    idx3, proj = pl.pallas_call(
        _proj_argmax_kernel,
        grid=(B, T // TT),
        in_specs=[
            pl.BlockSpec((1, H, TT), lambda b, t: (b, 0, t)),
            pl.BlockSpec((CD, H), lambda b, t: (0, 0)),
            pl.BlockSpec((CD,), lambda b, t: (0,)),
            pl.BlockSpec((CS, CD), lambda b, t: (0, 0)),
        ],
        out_specs=[
            pl.BlockSpec((1, 1, TT), lambda b, t: (b, 0, t)),
            pl.BlockSpec((1, CD, TT), lambda b, t: (b, 0, t)),
        ],
        out_shape=[
            jax.ShapeDtypeStruct((B, 1, T), jnp.int32),
            jax.ShapeDtypeStruct((B, CD, T), jnp.float32),
        ],
    )(hidden_state, W_in, b_in, codebook)

    qT, idx2 = _sc_gather(codebook.reshape(CS * CD), idx3)

    out, loss2 = pl.pallas_call(
        _out_loss_kernel,
        grid=(B, T // TTB),
        in_specs=[
            pl.BlockSpec((CD, TTB), lambda b, t: (0, b * (T // TTB) + t)),
            pl.BlockSpec((1, CD, TTB), lambda b, t: (b, 0, t)),
            pl.BlockSpec((H, CD), lambda b, t: (0, 0)),
            pl.BlockSpec((H,), lambda b, t: (0,)),
        ],
        out_specs=[
            pl.BlockSpec((1, H, TTB), lambda b, t: (b, 0, t)),
            pl.BlockSpec(memory_space=pltpu.SMEM, block_shape=(1, 1),
                         index_map=lambda b, t: (0, 0)),
        ],
        out_shape=[
            jax.ShapeDtypeStruct((B, H, T), jnp.float32),
            jax.ShapeDtypeStruct((1, 1), jnp.float32),
        ],
    )(qT, proj, W_out, b_out)

    loss = loss2[0, 0]
    return out, loss, loss, idx2, proj


def kernel(hidden_state, W_in, b_in, codebook, W_out, b_out):
    return _vq(hidden_state, W_in, b_in, codebook, W_out, b_out)
